# initial kernel scaffold (unmeasured)
import jax
import jax.numpy as jnp
from jax import lax
from jax.experimental import pallas as pl
from jax.experimental.pallas import tpu as pltpu


def kernel(
    x,
):
    def body(*refs):
        pass

    out_shape = jax.ShapeDtypeStruct(..., jnp.float32)
    return pl.pallas_call(body, out_shape=out_shape)(...)


# baseline (device time: 72269 ns/iter reference)
import jax
import jax.numpy as jnp
from jax import lax
from jax.experimental import pallas as pl
from jax.experimental.pallas import tpu as pltpu

N_DEV = 4
BLK = 256


def kernel(x):
    m, n = x.shape
    nblk = m // BLK

    def body(x_hbm, out_hbm, res_ref, in_bufs, comm_ref,
             in_sems, out_sem, send_sems, recv_sems):
        my = lax.axis_index("i")
        left = (my - 1) % N_DEV
        right = (my + 1) % N_DEV

        barrier_sem = pltpu.get_barrier_semaphore()
        for nbr in [left, right]:
            pl.semaphore_signal(
                barrier_sem, inc=1,
                device_id=(nbr,), device_id_type=pl.DeviceIdType.MESH,
            )
        pl.semaphore_wait(barrier_sem, 2)

        def in_copy(b, slot):
            return pltpu.make_async_copy(
                x_hbm.at[pl.ds(b * BLK, BLK), :],
                in_bufs.at[slot],
                in_sems.at[slot],
            )

        rows = lax.broadcasted_iota(jnp.int32, (BLK, BLK), 0)
        cols = lax.broadcasted_iota(jnp.int32, (BLK, BLK), 1)
        tri = (rows >= cols).astype(jnp.float32)

        in_copy(0, 0).start()

        def p1(b, carry):
            slot = lax.rem(b, 2)

            @pl.when(b + 1 < nblk)
            def _():
                in_copy(b + 1, lax.rem(b + 1, 2)).start()

            in_copy(b, slot).wait()
            blk = in_bufs[slot]
            cs = jnp.dot(tri, blk, preferred_element_type=jnp.float32)
            res_ref[pl.ds(b * BLK, BLK), :] = cs + carry
            return carry + cs[BLK - 1:BLK, :]

        total = lax.fori_loop(0, nblk, p1, jnp.zeros((1, n), jnp.float32))
        comm_ref[0, :, :] = total

        offset = jnp.zeros((1, n), jnp.float32)
        for h in range(N_DEV - 1):
            send_slot = h % 2
            recv_slot = (h + 1) % 2
            rdma = pltpu.make_async_remote_copy(
                src_ref=comm_ref.at[send_slot],
                dst_ref=comm_ref.at[recv_slot],
                send_sem=send_sems.at[send_slot],
                recv_sem=recv_sems.at[recv_slot],
                device_id=(right,),
                device_id_type=pl.DeviceIdType.MESH,
            )
            rdma.start()
            rdma.wait()
            origin = (my - h - 1) % N_DEV
            mask = (origin < my).astype(jnp.float32)
            offset = offset + comm_ref[recv_slot] * mask

        def p3(b, _):
            blk = res_ref[pl.ds(b * BLK, BLK), :]
            res_ref[pl.ds(b * BLK, BLK), :] = blk + offset
            return 0

        lax.fori_loop(0, nblk, p3, 0)

        out_copy = pltpu.make_async_copy(res_ref, out_hbm, out_sem)
        out_copy.start()
        out_copy.wait()

    return pl.pallas_call(
        body,
        out_shape=jax.ShapeDtypeStruct((m, n), jnp.float32),
        in_specs=[pl.BlockSpec(memory_space=pl.ANY)],
        out_specs=pl.BlockSpec(memory_space=pl.ANY),
        scratch_shapes=[
            pltpu.VMEM((m, n), jnp.float32),
            pltpu.VMEM((2, BLK, n), jnp.float32),
            pltpu.VMEM((2, 1, n), jnp.float32),
            pltpu.SemaphoreType.DMA((2,)),
            pltpu.SemaphoreType.DMA,
            pltpu.SemaphoreType.DMA((2,)),
            pltpu.SemaphoreType.DMA((2,)),
        ],
        compiler_params=pltpu.CompilerParams(
            collective_id=0, vmem_limit_bytes=60 * 1024 * 1024
        ),
    )(x)
